# baseline (speedup 1.0000x reference)
import functools
import numpy as np
import jax
import jax.numpy as jnp
from jax import lax
from jax.experimental import pallas as pl
from jax.experimental.pallas import tpu as pltpu
from jax.experimental.pallas import tpu_sc as plsc

N_BINS = 18
B = 32
T = 16384
NW = 16
TW = T // NW
NBINS_PAD = N_BINS + 2
ACC = NBINS_PAD * B

_INV_DELTA = np.float32(N_BINS / (2.0 * np.pi))
_PI = np.float32(np.pi)


def _binify_body(pha_ref, offs_ref):
    f = (pha_ref[...] + _PI) * _INV_DELTA
    idx = f.astype(jnp.int32)
    idx = jnp.minimum(idx, N_BINS)
    idx = jnp.where(f < 0.0, N_BINS, idx)
    offs_ref[...] = idx * B


def _sc_body(offs_hbm, ampT_hbm, out_hbm, offs_v, amp_v, acc_v, sem):
    w = lax.axis_index("s")

    pltpu.async_copy(
        offs_hbm.at[pl.ds(w * (B * TW), B * TW)], offs_v, sem
    )

    def zero(k, _):
        acc_v[pl.ds(k * 16, 16)] = jnp.zeros((16,), jnp.float32)
        return 0

    lax.fori_loop(0, B * ACC // 16, zero, 0)

    pltpu.make_async_copy(
        offs_hbm.at[pl.ds(0, B * TW)], offs_v, sem
    ).wait()
    pltpu.sync_copy(ampT_hbm.at[pl.ds(w * (TW * B), TW * B)], amp_v)

    lane_iota = lax.iota(jnp.int32, 16)

    def per_j(jj, _):
        obase = jj * TW
        base0 = lane_iota + jj * ACC
        base1 = base0 + 16

        @plsc.parallel_loop(0, TW // 16, unroll=2)
        def _(g):
            off_vec = offs_v[pl.ds(obase + g * 16, 16)]
            tbase = g * (16 * B)
            for u in range(16):
                off_splat = jnp.take_along_axis(
                    off_vec, jnp.full((16,), u, jnp.int32), axis=0,
                    mode="promise_in_bounds"
                )
                a0 = amp_v[pl.ds(tbase + u * B, 16)]
                a1 = amp_v[pl.ds(tbase + u * B + 16, 16)]
                plsc.addupdate_scatter(acc_v, [base0 + off_splat], a0)
                plsc.addupdate_scatter(acc_v, [base1 + off_splat], a1)

        return 0

    lax.fori_loop(0, B, per_j, 0)
    pltpu.sync_copy(acc_v, out_hbm.at[pl.ds(w * (B * ACC), B * ACC)])


@functools.partial(
    pl.kernel,
    out_type=jax.ShapeDtypeStruct((NW * B * ACC,), jnp.float32),
    mesh=plsc.VectorSubcoreMesh(
        core_axis_name="c", subcore_axis_name="s", num_cores=1, num_subcores=16
    ),
    scratch_types=[
        pltpu.VMEM((B * TW,), jnp.int32),
        pltpu.VMEM((TW * B,), jnp.float32),
        pltpu.VMEM((B * ACC,), jnp.float32),
        pltpu.SemaphoreType.DMA,
    ],
    compiler_params=pltpu.CompilerParams(needs_layout_passes=False),
)
def _sc_binsum(offs_hbm, ampT_hbm, out_hbm, offs_v, amp_v, acc_v, sem):
    _sc_body(offs_hbm, ampT_hbm, out_hbm, offs_v, amp_v, acc_v, sem)


def _entropy_body(sums_ref, out_ref):
    s = sums_ref[...]
    r = jnp.sum(s, axis=0)
    s18 = r[:, :N_BINS, :]
    tot = jnp.sum(s18, axis=1, keepdims=True)
    p = s18 / tot
    inv_log_n = np.float32(1.0 / np.log(float(N_BINS)))
    mi = 1.0 + inv_log_n * jnp.sum(p * jnp.log(p), axis=1)
    out_ref[...] = mi


@jax.jit
def kernel(pha, amp):
    offs = pl.pallas_call(
        _binify_body,
        out_shape=jax.ShapeDtypeStruct((B, T), jnp.int32),
    )(pha)
    offs_t = offs.reshape(B, NW, TW).swapaxes(0, 1).reshape(-1)
    sums = _sc_binsum(offs_t, amp.T.reshape(-1))
    mit = pl.pallas_call(
        _entropy_body,
        out_shape=jax.ShapeDtypeStruct((B, B), jnp.float32),
    )(sums.reshape(NW, B, NBINS_PAD, B))
    return mit.T


# speedup vs baseline: 2.7842x; 2.7842x over previous
import functools
import numpy as np
import jax
import jax.numpy as jnp
from jax import lax
from jax.experimental import pallas as pl
from jax.experimental.pallas import tpu as pltpu
from jax.experimental.pallas import tpu_sc as plsc

N_BINS = 18
B = 32
T = 16384
T_SC = 2048
T_TC = T - T_SC
NBINS_PAD = N_BINS + 2
ACC = NBINS_PAD * B

_INV_DELTA = np.float32(N_BINS / (2.0 * np.pi))
_PI = np.float32(np.pi)


def _sc_body(pha_hbm, ampT_hbm, out_hbm, pha_v, offs_v, amp_v, acc_v, sem):
    j = lax.axis_index("s") * 2 + lax.axis_index("c")

    pltpu.async_copy(ampT_hbm.at[pl.ds(0, T_SC * B)], amp_v, sem)
    pltpu.sync_copy(pha_hbm.at[pl.ds(j * T, T_SC)], pha_v)

    @plsc.parallel_loop(0, ACC // 16, unroll=8)
    def _zero(k):
        acc_v[pl.ds(k * 16, 16)] = jnp.zeros((16,), jnp.float32)

    @plsc.parallel_loop(0, T_SC // 16, unroll=4)
    def _binify(k):
        f = (pha_v[pl.ds(k * 16, 16)] + _PI) * _INV_DELTA
        idx = f.astype(jnp.int32)
        idx = jnp.minimum(idx, N_BINS)
        idx = jnp.where(f < 0.0, N_BINS, idx)
        offs_v[pl.ds(k * 16, 16)] = idx * B

    pltpu.make_async_copy(ampT_hbm.at[pl.ds(0, T_SC * B)], amp_v, sem).wait()

    lane_iota = lax.iota(jnp.int32, 16)
    base1 = lane_iota + 16

    @plsc.parallel_loop(0, T_SC // 16, unroll=2)
    def _accum(g):
        off_vec = offs_v[pl.ds(g * 16, 16)]
        tbase = g * (16 * B)
        for u in range(16):
            off_splat = jnp.take_along_axis(
                off_vec, jnp.full((16,), u, jnp.int32), axis=0,
                mode="promise_in_bounds"
            )
            a0 = amp_v[pl.ds(tbase + u * B, 16)]
            a1 = amp_v[pl.ds(tbase + u * B + 16, 16)]
            plsc.addupdate_scatter(acc_v, [lane_iota + off_splat], a0)
            plsc.addupdate_scatter(acc_v, [base1 + off_splat], a1)

    pltpu.sync_copy(acc_v, out_hbm.at[pl.ds(j * ACC, ACC)])


@functools.partial(
    pl.kernel,
    out_type=jax.ShapeDtypeStruct((B * ACC,), jnp.float32),
    mesh=plsc.VectorSubcoreMesh(
        core_axis_name="c", subcore_axis_name="s", num_cores=2, num_subcores=16
    ),
    scratch_types=[
        pltpu.VMEM((T_SC,), jnp.float32),
        pltpu.VMEM((T_SC,), jnp.int32),
        pltpu.VMEM((T_SC * B,), jnp.float32),
        pltpu.VMEM((ACC,), jnp.float32),
        pltpu.SemaphoreType.DMA,
    ],
    compiler_params=pltpu.CompilerParams(needs_layout_passes=False),
)
def _sc_binsum(pha_hbm, ampT_hbm, out_hbm, pha_v, offs_v, amp_v, acc_v, sem):
    _sc_body(pha_hbm, ampT_hbm, out_hbm, pha_v, offs_v, amp_v, acc_v, sem)


def _mm_body(cut_ref, pha_ref, amp_ref, out_ref):
    amp = amp_ref[...]
    cut = cut_ref[...]
    lows = cut[0, :N_BINS].reshape(N_BINS, 1)
    highs = cut[0, 1 : N_BINS + 1].reshape(N_BINS, 1)

    def per_j(jj, carry):
        pha_j = pha_ref[pl.ds(jj, 1), :]
        oh = ((lows < pha_j) & (pha_j < highs)).astype(jnp.float32)
        sums = lax.dot_general(
            oh, amp, (((1,), (1,)), ((), ())),
            preferred_element_type=jnp.float32,
        )
        out_ref[pl.ds(jj, 1), :, :] = sums.reshape(1, N_BINS, B)
        return carry

    lax.fori_loop(0, B, per_j, 0)


def _fin_body(sc_ref, tc_ref, out_ref):
    s = sc_ref[...][:, :N_BINS, :] + tc_ref[...]
    tot = jnp.sum(s, axis=1, keepdims=True)
    p = s / tot
    inv_log_n = np.float32(1.0 / np.log(float(N_BINS)))
    mi = 1.0 + inv_log_n * jnp.sum(p * jnp.log(p), axis=1)
    out_ref[...] = mi


@jax.jit
def kernel(pha, amp):
    cutoffs = jnp.linspace(-np.pi, np.pi, N_BINS + 1).astype(pha.dtype)
    cutoffs = cutoffs.reshape(1, N_BINS + 1)
    ampT_sc = amp[:, :T_SC].T.reshape(-1)
    sc_sums = _sc_binsum(pha.reshape(-1), ampT_sc)
    tc_sums = pl.pallas_call(
        _mm_body,
        out_shape=jax.ShapeDtypeStruct((B, N_BINS, B), jnp.float32),
    )(cutoffs, pha[:, T_SC:], amp[:, T_SC:])
    mit = pl.pallas_call(
        _fin_body,
        out_shape=jax.ShapeDtypeStruct((B, B), jnp.float32),
    )(sc_sums.reshape(B, NBINS_PAD, B), tc_sums)
    return mit.T
